# baseline (speedup 1.0000x reference)
import functools

import jax
import jax.numpy as jnp
from jax import lax
from jax.experimental import pallas as pl
from jax.experimental.pallas import tpu as pltpu
from jax.experimental.pallas import tpu_sc as plsc

_NW = 32
_G = 128
_NBUF = 4
_L = 16


def _gather_t(xT, table2, h_len, b_len, d_len):
    jb = b_len // _G
    jw = jb // _NW
    n_g = h_len * jw
    n_outer = n_g // _NBUF
    mesh = plsc.VectorSubcoreMesh(core_axis_name="c", subcore_axis_name="s")

    @functools.partial(
        pl.kernel,
        mesh=mesh,
        out_type=jax.ShapeDtypeStruct((h_len, d_len, b_len), jnp.float32),
        scratch_types=(
            [pltpu.VMEM((h_len, jw * _G), jnp.int32),
             pltpu.VMEM((_NBUF, _G), jnp.int32),
             pltpu.VMEM((_NBUF, _G, 128), jnp.float32),
             pltpu.VMEM((_NBUF, d_len, _G), jnp.float32)]
            + [pltpu.SemaphoreType.DMA] * (2 * _NBUF + 1)
        ),
        compiler_params=pltpu.CompilerParams(
            use_tc_tiling_on_sc=True, needs_layout_passes=False),
    )
    def body(xT_hbm, tab_hbm, out_hbm, idx_v, pair_v, g_v, t_v, *sems):
        gsem, osem, isem = sems[:_NBUF], sems[_NBUF:2 * _NBUF], sems[-1]
        wid = lax.axis_index("s") * 2 + lax.axis_index("c")
        col0 = wid * (jw * _G)
        pltpu.async_copy(xT_hbm.at[:, pl.ds(col0, jw * _G)], idx_v, isem).wait()

        iotas = [jax.lax.iota(jnp.int32, _L) + c * _L for c in range(8)]

        def prepare(g, s):
            h, jl = g // jw, g % jw
            for c in range(8):
                v = idx_v[h, pl.ds(jl * _G + c * _L, _L)]
                pair_v[s, pl.ds(c * _L, _L)] = v >> 1
            pltpu.async_copy(tab_hbm.at[pair_v.at[s]], g_v.at[s], gsem[s])

        def wait_gather(g, s):
            pltpu.make_async_copy(
                tab_hbm.at[pair_v.at[s]], g_v.at[s], gsem[s]).wait()

        def transpose(g, s):
            h, jl = g // jw, g % jw
            sels = []
            for c in range(8):
                v = idx_v[h, pl.ds(jl * _G + c * _L, _L)]
                sels.append((v & 1) << 6)

            def dbody(d, carry):
                for c in range(8):
                    vals = plsc.load_gather(g_v.at[s], [iotas[c], sels[c] + d])
                    t_v[s, d, pl.ds(c * _L, _L)] = vals
                return carry

            lax.fori_loop(0, d_len, dbody, 0)

        def store(g, s):
            h, jl = g // jw, g % jw
            pltpu.async_copy(
                t_v.at[s], out_hbm.at[h, :, pl.ds(col0 + jl * _G, _G)],
                osem[s])

        def wait_store(g, s):
            h, jl = g // jw, g % jw
            pltpu.make_async_copy(
                t_v.at[s], out_hbm.at[h, :, pl.ds(col0 + jl * _G, _G)],
                osem[s]).wait()

        for s in range(_NBUF):
            prepare(s, s)
        for s in range(_NBUF):
            wait_gather(s, s)
            transpose(s, s)
            store(s, s)
            prepare(s + _NBUF, s)

        def outer(o, carry):
            for s in range(_NBUF):
                g = o * _NBUF + s
                wait_gather(g, s)
                wait_store(g - _NBUF, s)
                transpose(g, s)
                store(g, s)
                prepare(g + _NBUF, s)
            return carry

        lax.fori_loop(1, n_outer - 1, outer, 0)

        for s in range(_NBUF):
            g = (n_outer - 1) * _NBUF + s
            wait_gather(g, s)
            wait_store(g - _NBUF, s)
            transpose(g, s)
            store(g, s)
        for s in range(_NBUF):
            wait_store((n_outer - 1) * _NBUF + s, s)

    return body(xT, table2)


def kernel(x, embedding_matrix):
    b_len, h_len = x.shape
    v_len, d_len = embedding_matrix.shape
    xT = x.astype(jnp.int32).T
    table2 = embedding_matrix.reshape(v_len // 2, 2 * d_len)
    out = _gather_t(xT, table2, h_len, b_len, d_len)
    return jnp.transpose(out, (2, 0, 1))


# speedup vs baseline: 1.4606x; 1.4606x over previous
import functools

import jax
import jax.numpy as jnp
from jax import lax
from jax.experimental import pallas as pl
from jax.experimental.pallas import tpu as pltpu
from jax.experimental.pallas import tpu_sc as plsc

_NW = 32
_G = 128
_NS = 4
_NP = 3
_NB = _NS * _NP


def _gather_flat(xf, table, n, d_len):
    per_w = n // _NW
    nb = per_w // _G
    n_rounds = nb // _NB
    rem = nb - n_rounds * _NB
    mesh = plsc.VectorSubcoreMesh(core_axis_name="c", subcore_axis_name="s")

    def buf(g):
        return (g % _NS) * _NP + (g // _NS) % _NP

    @functools.partial(
        pl.kernel,
        mesh=mesh,
        out_type=jax.ShapeDtypeStruct((n, d_len), jnp.float32),
        scratch_types=(
            [pltpu.VMEM((nb, _G), jnp.int32),
             pltpu.VMEM((_NB, _G, d_len), jnp.float32)]
            + [pltpu.SemaphoreType.DMA] * (2 * _NB + 1)
        ),
        compiler_params=pltpu.CompilerParams(
            use_tc_tiling_on_sc=False, needs_layout_passes=False),
    )
    def body(xf_hbm, tab_hbm, out_hbm, idx_v, g_v, *sems):
        gsem, osem, isem = sems[:_NB], sems[_NB:2 * _NB], sems[-1]
        wid = lax.axis_index("s") * 2 + lax.axis_index("c")
        row0 = wid * per_w
        pltpu.async_copy(xf_hbm.at[pl.ds(wid * nb, nb)], idx_v, isem).wait()

        def gather_fire(g, p):
            pltpu.async_copy(tab_hbm.at[idx_v.at[g]], g_v.at[p], gsem[p])

        def gather_wait(g, p):
            pltpu.make_async_copy(
                tab_hbm.at[idx_v.at[g]], g_v.at[p], gsem[p]).wait()

        def store_fire(g, p):
            pltpu.async_copy(
                g_v.at[p], out_hbm.at[pl.ds(row0 + g * _G, _G)], osem[p])

        def store_wait(g, p):
            pltpu.make_async_copy(
                g_v.at[p], out_hbm.at[pl.ds(row0 + g * _G, _G)],
                osem[p]).wait()

        def step(g, j, first=False, last=False):
            gather_wait(g, buf(j % _NB))
            if not first or j >= _NS:
                store_fire(g - _NS, buf((j - _NS) % _NB))
            if not first or j >= 2 * _NS:
                store_wait(g - 2 * _NS, buf((j - 2 * _NS) % _NB))
            if not last or j < rem - _NS:
                gather_fire(g + _NS, buf((j + _NS) % _NB))

        for j in range(_NS):
            gather_fire(j, buf(j))
        for j in range(_NB):
            step(j, j, first=True)

        def round_body(r, carry):
            for j in range(_NB):
                step(r * _NB + j, j, False, False)
            return carry

        lax.fori_loop(1, n_rounds, round_body, 0)

        for j in range(rem):
            step(n_rounds * _NB + j, j, False, True)
        for j in range(rem):
            g = n_rounds * _NB + j
            if j >= rem - _NS:
                store_fire(g, buf(j))
            else:
                store_wait(g, buf(j))
        for j in range(rem - _NS, rem):
            store_wait(n_rounds * _NB + j, buf(j))

    return body(xf, table)


def kernel(x, embedding_matrix):
    b_len, h_len = x.shape
    v_len, d_len = embedding_matrix.shape
    n = b_len * h_len
    xf = x.astype(jnp.int32).reshape(n // _G, _G)
    out = _gather_flat(xf, embedding_matrix, n, d_len)
    return out.reshape(b_len, h_len, d_len)
